# Initial kernel scaffold; baseline (speedup 1.0000x reference)
import jax
import jax.numpy as jnp
from jax.experimental import pallas as pl


def kernel(x, w):
    raise NotImplementedError("write your pallas kernel here")


# speedup vs baseline: 4.7576x; 4.7576x over previous
import functools

import jax
import jax.numpy as jnp
from jax import lax
from jax.experimental import pallas as pl
from jax.experimental.pallas import tpu as pltpu
from jax.experimental.pallas import tpu_sc as plsc

C_IN = 100000
D = 64
B_TOTAL = 16384 * 200

NC = 2
NS = 16
NW = NC * NS

IDX_J = 128
K = 4
SLAB = IDX_J * K
ROWS_TOTAL = B_TOTAL // IDX_J
ROWS_PER_W = ROWS_TOTAL // NW
N_SLABS = ROWS_PER_W // K


def _sc_gather(x_rows, table):
  mesh = plsc.VectorSubcoreMesh(core_axis_name="c", subcore_axis_name="s")

  @functools.partial(
      pl.kernel,
      mesh=mesh,
      out_type=jax.ShapeDtypeStruct((B_TOTAL, D), jnp.float32),
      scratch_types=[
          pltpu.VMEM((K, IDX_J), jnp.int32),
          pltpu.VMEM((SLAB, D), jnp.float32),
          pltpu.SemaphoreType.DMA,
      ],
      compiler_params=pltpu.CompilerParams(use_tc_tiling_on_sc=False),
  )
  def k(x_hbm, table_hbm, out_hbm, idx_v, rows_v, sem):
    wid = lax.axis_index("s") * NC + lax.axis_index("c")
    base_row = wid * ROWS_PER_W

    def step(g, _):
      row_off = base_row + g * K
      pltpu.sync_copy(x_hbm.at[pl.ds(row_off, K)], idx_v)
      copies = []
      for j in range(K):
        copies.append(
            pltpu.async_copy(
                table_hbm.at[idx_v.at[j]],
                rows_v.at[pl.ds(j * IDX_J, IDX_J)],
                sem,
            )
        )
      for c in copies:
        c.wait()
      pltpu.sync_copy(rows_v, out_hbm.at[pl.ds(row_off * IDX_J, SLAB)])
      return 0

    lax.fori_loop(0, N_SLABS, step, 0)

  return k(x_rows, table)


def kernel(x, w):
  x_rows = x.reshape(ROWS_TOTAL, IDX_J)
  out = _sc_gather(x_rows, w)
  return lax.stop_gradient(out.reshape(x.shape[0], x.shape[1], D))


# speedup vs baseline: 5.1709x; 1.0869x over previous
import functools

import jax
import jax.numpy as jnp
from jax import lax
from jax.experimental import pallas as pl
from jax.experimental.pallas import tpu as pltpu
from jax.experimental.pallas import tpu_sc as plsc

C_IN = 100000
D = 64
B_TOTAL = 16384 * 200

NC = 2
NS = 16
NW = NC * NS

IDX_J = 128
K = 4
SLAB = IDX_J * K
ROWS_TOTAL = B_TOTAL // IDX_J
ROWS_PER_W = ROWS_TOTAL // NW
N_SLABS = ROWS_PER_W // K

NIDX = 4
NROW = 2


def _sc_gather(x_rows, table):
  mesh = plsc.VectorSubcoreMesh(core_axis_name="c", subcore_axis_name="s")

  @functools.partial(
      pl.kernel,
      mesh=mesh,
      out_type=jax.ShapeDtypeStruct((B_TOTAL, D), jnp.float32),
      scratch_types=[
          pltpu.VMEM((NIDX, K, IDX_J), jnp.int32),
          pltpu.VMEM((NROW, SLAB, D), jnp.float32),
          [pltpu.SemaphoreType.DMA] * NIDX,
          [pltpu.SemaphoreType.DMA] * NROW,
          [pltpu.SemaphoreType.DMA] * NROW,
      ],
      compiler_params=pltpu.CompilerParams(use_tc_tiling_on_sc=False),
  )
  def k(x_hbm, table_hbm, out_hbm, idx_v, rows_v, idx_sems, g_sems, o_sems):
    wid = lax.axis_index("s") * NC + lax.axis_index("c")
    base_row = wid * ROWS_PER_W

    def idx_fire(g, q):
      gw = lax.rem(g, N_SLABS) if not isinstance(g, int) else g % N_SLABS
      return pltpu.async_copy(
          x_hbm.at[pl.ds(base_row + gw * K, K)], idx_v.at[q], idx_sems[q])

    def idx_wait(q):
      pltpu.make_async_copy(
          x_hbm.at[pl.ds(base_row, K)], idx_v.at[q], idx_sems[q]).wait()

    def gathers_fire(g, b, q):
      for j in range(K):
        pltpu.async_copy(
            table_hbm.at[idx_v.at[q, j]],
            rows_v.at[b, pl.ds(j * IDX_J, IDX_J)],
            g_sems[b])

    def gathers_drain(b, q):
      for j in range(K):
        pltpu.make_async_copy(
            table_hbm.at[idx_v.at[q, j]],
            rows_v.at[b, pl.ds(j * IDX_J, IDX_J)],
            g_sems[b]).wait()

    def out_fire(g, b):
      pltpu.async_copy(
          rows_v.at[b],
          out_hbm.at[pl.ds((base_row + g * K) * IDX_J, SLAB)],
          o_sems[b])

    def out_wait(b):
      pltpu.make_async_copy(
          rows_v.at[b],
          out_hbm.at[pl.ds(base_row * IDX_J, SLAB)],
          o_sems[b]).wait()

    idx_fire(0, 0)
    idx_fire(1, 1)
    idx_fire(2, 2)
    idx_wait(0)
    gathers_fire(0, 0, 0)
    idx_fire(3, 3)
    idx_wait(1)
    gathers_fire(1, 1, 1)
    gathers_drain(0, 0)
    out_fire(0, 0)
    idx_fire(4, 0)

    @pl.loop(2, N_SLABS - 2, step=NIDX)
    def _(og):
      for u in range(NIDX):
        g = og + u
        b = (2 + u) % NROW
        q = (2 + u) % NIDX
        pb = (1 + u) % NROW
        pq = (1 + u) % NIDX
        out_wait(b)
        idx_wait(q)
        gathers_fire(g, b, q)
        gathers_drain(pb, pq)
        out_fire(g - 1, pb)
        idx_fire(g + 3, pq)

    out_wait(0)
    idx_wait(2)
    gathers_fire(N_SLABS - 2, 0, 2)
    gathers_drain(1, 1)
    out_fire(N_SLABS - 3, 1)

    out_wait(1)
    idx_wait(3)
    gathers_fire(N_SLABS - 1, 1, 3)
    gathers_drain(0, 2)
    out_fire(N_SLABS - 2, 0)

    gathers_drain(1, 3)
    out_fire(N_SLABS - 1, 1)
    out_wait(0)
    out_wait(1)
    idx_wait(0)

  return k(x_rows, table)


def kernel(x, w):
  x_rows = x.reshape(ROWS_TOTAL, IDX_J)
  out = _sc_gather(x_rows, w)
  return lax.stop_gradient(out.reshape(x.shape[0], x.shape[1], D))


# speedup vs baseline: 5.1831x; 1.0024x over previous
import functools

import jax
import jax.numpy as jnp
from jax import lax
from jax.experimental import pallas as pl
from jax.experimental.pallas import tpu as pltpu
from jax.experimental.pallas import tpu_sc as plsc

C_IN = 100000
D = 64
BATCH = 16384
HIST = 200

NC = 2
NS = 16
NW = NC * NS

BPW = BATCH // NW
SB = 2
SPLITS = ((0, 128), (128, 72))
N_SLABS = BPW // SB

NIDX = 4
NROW = 2


def _sc_gather(x, table):
  mesh = plsc.VectorSubcoreMesh(core_axis_name="c", subcore_axis_name="s")

  @functools.partial(
      pl.kernel,
      mesh=mesh,
      out_type=jax.ShapeDtypeStruct((BATCH, HIST, D), jnp.float32),
      scratch_types=[
          pltpu.VMEM((NIDX, SB, HIST), jnp.int32),
          pltpu.VMEM((NROW, SB, HIST, D), jnp.float32),
          [pltpu.SemaphoreType.DMA] * NIDX,
          [pltpu.SemaphoreType.DMA] * NROW,
          [pltpu.SemaphoreType.DMA] * NROW,
      ],
      compiler_params=pltpu.CompilerParams(use_tc_tiling_on_sc=False),
  )
  def k(x_hbm, table_hbm, out_hbm, idx_v, rows_v, idx_sems, g_sems, o_sems):
    wid = lax.axis_index("s") * NC + lax.axis_index("c")
    bbase = wid * BPW

    def idx_fire(g, q):
      gw = lax.rem(g, N_SLABS) if not isinstance(g, int) else g % N_SLABS
      return pltpu.async_copy(
          x_hbm.at[pl.ds(bbase + gw * SB, SB)], idx_v.at[q], idx_sems[q])

    def idx_wait(q):
      pltpu.make_async_copy(
          x_hbm.at[pl.ds(bbase, SB)], idx_v.at[q], idx_sems[q]).wait()

    def gathers_fire(g, b, q):
      for i in range(SB):
        for off, ln in SPLITS:
          pltpu.async_copy(
              table_hbm.at[idx_v.at[q, i, pl.ds(off, ln)]],
              rows_v.at[b, i, pl.ds(off, ln)],
              g_sems[b])

    def gathers_drain(b, q):
      for i in range(SB):
        for off, ln in SPLITS:
          pltpu.make_async_copy(
              table_hbm.at[idx_v.at[q, i, pl.ds(off, ln)]],
              rows_v.at[b, i, pl.ds(off, ln)],
              g_sems[b]).wait()

    def out_fire(g, b):
      pltpu.async_copy(
          rows_v.at[b], out_hbm.at[pl.ds(bbase + g * SB, SB)], o_sems[b])

    def out_wait(b):
      pltpu.make_async_copy(
          rows_v.at[b], out_hbm.at[pl.ds(bbase, SB)], o_sems[b]).wait()

    idx_fire(0, 0)
    idx_fire(1, 1)
    idx_fire(2, 2)
    idx_wait(0)
    gathers_fire(0, 0, 0)
    idx_fire(3, 3)
    idx_wait(1)
    gathers_fire(1, 1, 1)
    gathers_drain(0, 0)
    out_fire(0, 0)
    idx_fire(4, 0)

    @pl.loop(2, N_SLABS - 2, step=NIDX)
    def _(og):
      for u in range(NIDX):
        g = og + u
        b = (2 + u) % NROW
        q = (2 + u) % NIDX
        pb = (1 + u) % NROW
        pq = (1 + u) % NIDX
        out_wait(b)
        idx_wait(q)
        gathers_fire(g, b, q)
        gathers_drain(pb, pq)
        out_fire(g - 1, pb)
        idx_fire(g + 3, pq)

    out_wait(0)
    idx_wait(2)
    gathers_fire(N_SLABS - 2, 0, 2)
    gathers_drain(1, 1)
    out_fire(N_SLABS - 3, 1)

    out_wait(1)
    idx_wait(3)
    gathers_fire(N_SLABS - 1, 1, 3)
    gathers_drain(0, 2)
    out_fire(N_SLABS - 2, 0)

    gathers_drain(1, 3)
    out_fire(N_SLABS - 1, 1)
    out_wait(0)
    out_wait(1)
    idx_wait(0)

  return k(x, table)


def kernel(x, w):
  return lax.stop_gradient(_sc_gather(x, w))


# speedup vs baseline: 5.2270x; 1.0085x over previous
import functools

import jax
import jax.numpy as jnp
from jax import lax
from jax.experimental import pallas as pl
from jax.experimental.pallas import tpu as pltpu
from jax.experimental.pallas import tpu_sc as plsc

C_IN = 100000
D = 64
BATCH = 16384
HIST = 200

NC = 2
NS = 16
NW = NC * NS

BPW = BATCH // NW
SB = 2
SPLITS = ((0, 128), (128, 72))
N_SLABS = BPW // SB
NSLAB_ALL = BATCH // SB

NIDX = 4
NROW = 2


def _sc_gather(x, table):
  mesh = plsc.VectorSubcoreMesh(core_axis_name="c", subcore_axis_name="s")

  @functools.partial(
      pl.kernel,
      mesh=mesh,
      out_type=jax.ShapeDtypeStruct((NSLAB_ALL, HIST, 2 * D), jnp.float32),
      scratch_types=[
          pltpu.VMEM((NIDX, SB, HIST), jnp.int32),
          pltpu.VMEM((NROW, SB, HIST, D), jnp.float32),
          [pltpu.SemaphoreType.DMA] * NIDX,
          [pltpu.SemaphoreType.DMA] * NROW,
          [pltpu.SemaphoreType.DMA] * NROW,
      ],
      compiler_params=pltpu.CompilerParams(use_tc_tiling_on_sc=False),
  )
  def k(x_hbm, table_hbm, out_hbm, idx_v, rows_v, idx_sems, g_sems, o_sems):
    wid = lax.axis_index("s") * NC + lax.axis_index("c")
    bbase = wid * BPW
    sbase = wid * N_SLABS

    def idx_fire(g, q):
      gw = lax.rem(g, N_SLABS) if not isinstance(g, int) else g % N_SLABS
      return pltpu.async_copy(
          x_hbm.at[pl.ds(bbase + gw * SB, SB)], idx_v.at[q], idx_sems[q])

    def idx_wait(q):
      pltpu.make_async_copy(
          x_hbm.at[pl.ds(bbase, SB)], idx_v.at[q], idx_sems[q]).wait()

    def gathers_fire(g, b, q):
      for e in range(SB):
        for off, ln in SPLITS:
          pltpu.async_copy(
              table_hbm.at[idx_v.at[q, e, pl.ds(off, ln)]],
              rows_v.at[b, e, pl.ds(off, ln)],
              g_sems[b])

    def gathers_drain(b, q):
      for e in range(SB):
        for off, ln in SPLITS:
          pltpu.make_async_copy(
              table_hbm.at[idx_v.at[q, e, pl.ds(off, ln)]],
              rows_v.at[b, e, pl.ds(off, ln)],
              g_sems[b]).wait()

    def out_fire(g, b):
      for e in range(SB):
        pltpu.async_copy(
            rows_v.at[b, e],
            out_hbm.at[sbase + g, pl.ds(0, HIST), pl.ds(e * D, D)],
            o_sems[b])

    def out_wait(b):
      for e in range(SB):
        pltpu.make_async_copy(
            rows_v.at[b, e],
            out_hbm.at[sbase, pl.ds(0, HIST), pl.ds(e * D, D)],
            o_sems[b]).wait()

    idx_fire(0, 0)
    idx_fire(1, 1)
    idx_fire(2, 2)
    idx_wait(0)
    gathers_fire(0, 0, 0)
    idx_fire(3, 3)
    idx_wait(1)
    gathers_fire(1, 1, 1)
    gathers_drain(0, 0)
    out_fire(0, 0)
    idx_fire(4, 0)

    @pl.loop(2, N_SLABS - 2, step=NIDX)
    def _(og):
      for u in range(NIDX):
        g = og + u
        b = (2 + u) % NROW
        q = (2 + u) % NIDX
        pb = (1 + u) % NROW
        pq = (1 + u) % NIDX
        out_wait(b)
        idx_wait(q)
        gathers_fire(g, b, q)
        gathers_drain(pb, pq)
        out_fire(g - 1, pb)
        idx_fire(g + 3, pq)

    out_wait(0)
    idx_wait(2)
    gathers_fire(N_SLABS - 2, 0, 2)
    gathers_drain(1, 1)
    out_fire(N_SLABS - 3, 1)

    out_wait(1)
    idx_wait(3)
    gathers_fire(N_SLABS - 1, 1, 3)
    gathers_drain(0, 2)
    out_fire(N_SLABS - 2, 0)

    gathers_drain(1, 3)
    out_fire(N_SLABS - 1, 1)
    out_wait(0)
    out_wait(1)
    idx_wait(0)

  return k(x, table)


SBLK = 64


HB = 40


def _tc_finish(packed):
  def body(in_ref, out_ref):
    x = in_ref[...]
    sl = lax.broadcasted_iota(jnp.int32, (SBLK, SB * SBLK), 0)
    bl = lax.broadcasted_iota(jnp.int32, (SBLK, SB * SBLK), 1)
    acc = None
    for e in range(SB):
      onehot = (bl == SB * sl + e).astype(jnp.float32)
      xe = x[:, :, e * D:(e + 1) * D]
      z = lax.dot_general(
          xe, onehot, (((0,), (0,)), ((), ())),
          precision=lax.Precision.HIGHEST,
          preferred_element_type=jnp.float32)
      acc = z if acc is None else acc + z
    out_ref[...] = acc

  return pl.pallas_call(
      body,
      grid=(NSLAB_ALL // SBLK, HIST // HB),
      in_specs=[pl.BlockSpec((SBLK, HB, 2 * D), lambda i, j: (i, j, 0))],
      out_specs=pl.BlockSpec((HB, D, SB * SBLK), lambda i, j: (j, 0, i)),
      out_shape=jax.ShapeDtypeStruct((HIST, D, BATCH), jnp.float32),
  )(packed)


def kernel(x, w):
  packed = _sc_gather(x, w)
  out_t = _tc_finish(packed)
  return lax.stop_gradient(out_t.transpose(2, 0, 1))


# speedup vs baseline: 5.8319x; 1.1157x over previous
import functools

import jax
import jax.numpy as jnp
from jax import lax
from jax.experimental import pallas as pl
from jax.experimental.pallas import tpu as pltpu
from jax.experimental.pallas import tpu_sc as plsc

C_IN = 100000
D = 64
BATCH = 16384
HIST = 200

NC = 2
NS = 16
NW = NC * NS

BPW = BATCH // NW
SB = 2
SPLITS = ((0, 128), (128, 72))
N_SLABS = BPW // SB
NSLAB_ALL = BATCH // SB

NIDX = 4
NROW = 2


def _sc_gather(x, table):
  mesh = plsc.VectorSubcoreMesh(core_axis_name="c", subcore_axis_name="s")

  @functools.partial(
      pl.kernel,
      mesh=mesh,
      out_type=jax.ShapeDtypeStruct((NSLAB_ALL, HIST, 2 * D), jnp.float32),
      scratch_types=[
          pltpu.VMEM((NIDX, SB, HIST), jnp.int32),
          pltpu.VMEM((NROW, SB, HIST, D), jnp.float32),
          [pltpu.SemaphoreType.DMA] * NIDX,
          [pltpu.SemaphoreType.DMA] * NROW,
          [pltpu.SemaphoreType.DMA] * NROW,
      ],
      compiler_params=pltpu.CompilerParams(use_tc_tiling_on_sc=False),
  )
  def k(x_hbm, table_hbm, out_hbm, idx_v, rows_v, idx_sems, g_sems, o_sems):
    wid = lax.axis_index("s") * NC + lax.axis_index("c")
    bbase = wid * BPW
    sbase = wid * N_SLABS

    def idx_fire(g, q):
      gw = lax.rem(g, N_SLABS) if not isinstance(g, int) else g % N_SLABS
      return pltpu.async_copy(
          x_hbm.at[pl.ds(bbase + gw * SB, SB)], idx_v.at[q], idx_sems[q])

    def idx_wait(q):
      pltpu.make_async_copy(
          x_hbm.at[pl.ds(bbase, SB)], idx_v.at[q], idx_sems[q]).wait()

    def gathers_fire(g, b, q):
      for e in range(SB):
        for off, ln in SPLITS:
          pltpu.async_copy(
              table_hbm.at[idx_v.at[q, e, pl.ds(off, ln)]],
              rows_v.at[b, e, pl.ds(off, ln)],
              g_sems[b])

    def gathers_drain(b, q):
      for e in range(SB):
        for off, ln in SPLITS:
          pltpu.make_async_copy(
              table_hbm.at[idx_v.at[q, e, pl.ds(off, ln)]],
              rows_v.at[b, e, pl.ds(off, ln)],
              g_sems[b]).wait()

    def out_fire(g, b):
      for e in range(SB):
        pltpu.async_copy(
            rows_v.at[b, e],
            out_hbm.at[sbase + g, pl.ds(0, HIST), pl.ds(e * D, D)],
            o_sems[b])

    def out_wait(b):
      for e in range(SB):
        pltpu.make_async_copy(
            rows_v.at[b, e],
            out_hbm.at[sbase, pl.ds(0, HIST), pl.ds(e * D, D)],
            o_sems[b]).wait()

    idx_fire(0, 0)
    idx_fire(1, 1)
    idx_fire(2, 2)
    idx_wait(0)
    gathers_fire(0, 0, 0)
    idx_fire(3, 3)
    idx_wait(1)
    gathers_fire(1, 1, 1)
    gathers_drain(0, 0)
    out_fire(0, 0)
    idx_fire(4, 0)

    @pl.loop(2, N_SLABS - 2, step=NIDX)
    def _(og):
      for u in range(NIDX):
        g = og + u
        b = (2 + u) % NROW
        q = (2 + u) % NIDX
        pb = (1 + u) % NROW
        pq = (1 + u) % NIDX
        out_wait(b)
        idx_wait(q)
        gathers_fire(g, b, q)
        gathers_drain(pb, pq)
        out_fire(g - 1, pb)
        idx_fire(g + 3, pq)

    out_wait(0)
    idx_wait(2)
    gathers_fire(N_SLABS - 2, 0, 2)
    gathers_drain(1, 1)
    out_fire(N_SLABS - 3, 1)

    out_wait(1)
    idx_wait(3)
    gathers_fire(N_SLABS - 1, 1, 3)
    gathers_drain(0, 2)
    out_fire(N_SLABS - 2, 0)

    gathers_drain(1, 3)
    out_fire(N_SLABS - 1, 1)
    out_wait(0)
    out_wait(1)
    idx_wait(0)

  return k(x, table)


SBLK = 128


HB = 40


def _tc_finish(packed):
  def body(in_ref, out_ref):
    x = in_ref[...]
    sl = lax.broadcasted_iota(jnp.int32, (SBLK, SB * SBLK), 0)
    bl = lax.broadcasted_iota(jnp.int32, (SBLK, SB * SBLK), 1)
    acc = None
    for e in range(SB):
      onehot = (bl == SB * sl + e).astype(jnp.float32)
      xe = x[:, :, e * D:(e + 1) * D]
      hi = xe.astype(jnp.bfloat16).astype(jnp.float32)
      r = xe - hi
      mid = r.astype(jnp.bfloat16).astype(jnp.float32)
      lo = r - mid
      for part in (hi, mid, lo):
        z = lax.dot_general(
            part, onehot, (((0,), (0,)), ((), ())),
            preferred_element_type=jnp.float32)
        acc = z if acc is None else acc + z
    out_ref[...] = acc

  return pl.pallas_call(
      body,
      grid=(NSLAB_ALL // SBLK, HIST // HB),
      in_specs=[pl.BlockSpec((SBLK, HB, 2 * D), lambda i, j: (i, j, 0))],
      out_specs=pl.BlockSpec((HB, D, SB * SBLK), lambda i, j: (j, 0, i)),
      out_shape=jax.ShapeDtypeStruct((HIST, D, BATCH), jnp.float32),
  )(packed)


def kernel(x, w):
  packed = _sc_gather(x, w)
  out_t = _tc_finish(packed)
  return lax.stop_gradient(out_t.transpose(2, 0, 1))


# speedup vs baseline: 7.7625x; 1.3310x over previous
import functools

import jax
import jax.numpy as jnp
from jax import lax
from jax.experimental import pallas as pl
from jax.experimental.pallas import tpu as pltpu
from jax.experimental.pallas import tpu_sc as plsc

C_IN = 100000
D = 64
BATCH = 16384
HIST = 200

NC = 2
NS = 16
NW = NC * NS

BPW = BATCH // NW
SB = 2
SPLITS = ((0, 128), (128, 72))
N_SLABS = BPW // SB
NSLAB_ALL = BATCH // SB

NIDX = 4
NROW = 2


def _sc_gather(x, table):
  mesh = plsc.VectorSubcoreMesh(core_axis_name="c", subcore_axis_name="s")

  @functools.partial(
      pl.kernel,
      mesh=mesh,
      out_type=jax.ShapeDtypeStruct((NSLAB_ALL, HIST, 2 * D), jnp.float32),
      scratch_types=[
          pltpu.VMEM((NIDX, SB, HIST), jnp.int32),
          pltpu.VMEM((NROW, SB, HIST, D), jnp.float32),
          [pltpu.SemaphoreType.DMA] * NIDX,
          [pltpu.SemaphoreType.DMA] * NROW,
          [pltpu.SemaphoreType.DMA] * NROW,
      ],
      compiler_params=pltpu.CompilerParams(use_tc_tiling_on_sc=False),
  )
  def k(x_hbm, table_hbm, out_hbm, idx_v, rows_v, idx_sems, g_sems, o_sems):
    wid = lax.axis_index("s") * NC + lax.axis_index("c")
    bbase = wid * BPW
    sbase = wid * N_SLABS

    def idx_fire(g, q):
      gw = lax.rem(g, N_SLABS) if not isinstance(g, int) else g % N_SLABS
      return pltpu.async_copy(
          x_hbm.at[pl.ds(bbase + gw * SB, SB)], idx_v.at[q], idx_sems[q])

    def idx_wait(q):
      pltpu.make_async_copy(
          x_hbm.at[pl.ds(bbase, SB)], idx_v.at[q], idx_sems[q]).wait()

    def gathers_fire(g, b, q):
      for e in range(SB):
        for off, ln in SPLITS:
          pltpu.async_copy(
              table_hbm.at[idx_v.at[q, e, pl.ds(off, ln)]],
              rows_v.at[b, e, pl.ds(off, ln)],
              g_sems[b])

    def gathers_drain(b, q):
      for e in range(SB):
        for off, ln in SPLITS:
          pltpu.make_async_copy(
              table_hbm.at[idx_v.at[q, e, pl.ds(off, ln)]],
              rows_v.at[b, e, pl.ds(off, ln)],
              g_sems[b]).wait()

    def out_fire(g, b):
      for e in range(SB):
        pltpu.async_copy(
            rows_v.at[b, e],
            out_hbm.at[sbase + g, pl.ds(0, HIST), pl.ds(e * D, D)],
            o_sems[b])

    def out_wait(b):
      for e in range(SB):
        pltpu.make_async_copy(
            rows_v.at[b, e],
            out_hbm.at[sbase, pl.ds(0, HIST), pl.ds(e * D, D)],
            o_sems[b]).wait()

    idx_fire(0, 0)
    idx_fire(1, 1)
    idx_fire(2, 2)
    idx_wait(0)
    gathers_fire(0, 0, 0)
    idx_fire(3, 3)
    idx_wait(1)
    gathers_fire(1, 1, 1)
    gathers_drain(0, 0)
    out_fire(0, 0)
    idx_fire(4, 0)

    @pl.loop(2, N_SLABS - 2, step=NIDX)
    def _(og):
      for u in range(NIDX):
        g = og + u
        b = (2 + u) % NROW
        q = (2 + u) % NIDX
        pb = (1 + u) % NROW
        pq = (1 + u) % NIDX
        out_wait(b)
        idx_wait(q)
        gathers_fire(g, b, q)
        gathers_drain(pb, pq)
        out_fire(g - 1, pb)
        idx_fire(g + 3, pq)

    out_wait(0)
    idx_wait(2)
    gathers_fire(N_SLABS - 2, 0, 2)
    gathers_drain(1, 1)
    out_fire(N_SLABS - 3, 1)

    out_wait(1)
    idx_wait(3)
    gathers_fire(N_SLABS - 1, 1, 3)
    gathers_drain(0, 2)
    out_fire(N_SLABS - 2, 0)

    gathers_drain(1, 3)
    out_fire(N_SLABS - 1, 1)
    out_wait(0)
    out_wait(1)
    idx_wait(0)

  return k(x, table)


SBLK = 128


HB = 40


def _tc_finish(packed):
  def body(in_ref, out_ref):
    x = in_ref[...]
    sl = lax.broadcasted_iota(jnp.int32, (SBLK, SB * SBLK), 0)
    bl = lax.broadcasted_iota(jnp.int32, (SBLK, SB * SBLK), 1)
    acc = None
    for e in range(SB):
      onehot = (bl == SB * sl + e).astype(jnp.float32)
      xe = x[:, :, e * D:(e + 1) * D]
      hi = xe.astype(jnp.bfloat16).astype(jnp.float32)
      lo = xe - hi
      for part in (hi, lo):
        z = lax.dot_general(
            part, onehot, (((0,), (0,)), ((), ())),
            preferred_element_type=jnp.float32)
        acc = z if acc is None else acc + z
    out_ref[...] = acc

  return pl.pallas_call(
      body,
      grid=(NSLAB_ALL // SBLK, HIST // HB),
      in_specs=[pl.BlockSpec((SBLK, HB, 2 * D), lambda i, j: (i, j, 0))],
      out_specs=pl.BlockSpec((HB, D, SB * SBLK), lambda i, j: (j, 0, i)),
      out_shape=jax.ShapeDtypeStruct((HIST, D, BATCH), jnp.float32),
  )(packed)


def kernel(x, w):
  packed = _sc_gather(x, w)
  out_t = _tc_finish(packed)
  return lax.stop_gradient(out_t.transpose(2, 0, 1))


# speedup vs baseline: 9.4313x; 1.2150x over previous
import functools

import jax
import jax.numpy as jnp
from jax import lax
from jax.experimental import pallas as pl
from jax.experimental.pallas import tpu as pltpu
from jax.experimental.pallas import tpu_sc as plsc

C_IN = 100000
D = 64
BATCH = 16384
HIST = 200

NC = 2
NS = 16
NW = NC * NS

BPW = BATCH // NW
SB = 2
SPLITS = ((0, 128), (128, 72))
N_SLABS = BPW // SB
NSLAB_ALL = BATCH // SB

NIDX = 4
NROW = 2


def _sc_gather(x, table):
  mesh = plsc.VectorSubcoreMesh(core_axis_name="c", subcore_axis_name="s")

  @functools.partial(
      pl.kernel,
      mesh=mesh,
      out_type=jax.ShapeDtypeStruct((NSLAB_ALL, HIST, 2 * D), jnp.float32),
      scratch_types=[
          pltpu.VMEM((NIDX, SB, HIST), jnp.int32),
          pltpu.VMEM((NROW, SB, HIST, D), jnp.float32),
          [pltpu.SemaphoreType.DMA] * NIDX,
          [pltpu.SemaphoreType.DMA] * NROW,
          [pltpu.SemaphoreType.DMA] * NROW,
      ],
      compiler_params=pltpu.CompilerParams(use_tc_tiling_on_sc=False),
  )
  def k(x_hbm, table_hbm, out_hbm, idx_v, rows_v, idx_sems, g_sems, o_sems):
    wid = lax.axis_index("s") * NC + lax.axis_index("c")
    bbase = wid * BPW
    sbase = wid * N_SLABS

    def idx_fire(g, q):
      gw = lax.rem(g, N_SLABS) if not isinstance(g, int) else g % N_SLABS
      return pltpu.async_copy(
          x_hbm.at[pl.ds(bbase + gw * SB, SB)], idx_v.at[q], idx_sems[q])

    def idx_wait(q):
      pltpu.make_async_copy(
          x_hbm.at[pl.ds(bbase, SB)], idx_v.at[q], idx_sems[q]).wait()

    def gathers_fire(g, b, q):
      for e in range(SB):
        for off, ln in SPLITS:
          pltpu.async_copy(
              table_hbm.at[idx_v.at[q, e, pl.ds(off, ln)]],
              rows_v.at[b, e, pl.ds(off, ln)],
              g_sems[b])

    def gathers_drain(b, q):
      for e in range(SB):
        for off, ln in SPLITS:
          pltpu.make_async_copy(
              table_hbm.at[idx_v.at[q, e, pl.ds(off, ln)]],
              rows_v.at[b, e, pl.ds(off, ln)],
              g_sems[b]).wait()

    def out_fire(g, b):
      for e in range(SB):
        pltpu.async_copy(
            rows_v.at[b, e],
            out_hbm.at[sbase + g, pl.ds(0, HIST), pl.ds(e * D, D)],
            o_sems[b])

    def out_wait(b):
      for e in range(SB):
        pltpu.make_async_copy(
            rows_v.at[b, e],
            out_hbm.at[sbase, pl.ds(0, HIST), pl.ds(e * D, D)],
            o_sems[b]).wait()

    idx_fire(0, 0)
    idx_fire(1, 1)
    idx_fire(2, 2)
    idx_wait(0)
    gathers_fire(0, 0, 0)
    idx_fire(3, 3)
    idx_wait(1)
    gathers_fire(1, 1, 1)
    gathers_drain(0, 0)
    out_fire(0, 0)
    idx_fire(4, 0)

    @pl.loop(2, N_SLABS - 2, step=NIDX)
    def _(og):
      for u in range(NIDX):
        g = og + u
        b = (2 + u) % NROW
        q = (2 + u) % NIDX
        pb = (1 + u) % NROW
        pq = (1 + u) % NIDX
        out_wait(b)
        idx_wait(q)
        gathers_fire(g, b, q)
        gathers_drain(pb, pq)
        out_fire(g - 1, pb)
        idx_fire(g + 3, pq)

    out_wait(0)
    idx_wait(2)
    gathers_fire(N_SLABS - 2, 0, 2)
    gathers_drain(1, 1)
    out_fire(N_SLABS - 3, 1)

    out_wait(1)
    idx_wait(3)
    gathers_fire(N_SLABS - 1, 1, 3)
    gathers_drain(0, 2)
    out_fire(N_SLABS - 2, 0)

    gathers_drain(1, 3)
    out_fire(N_SLABS - 1, 1)
    out_wait(0)
    out_wait(1)
    idx_wait(0)

  return k(x, table)


SBLK = 128


HB = 40


def _tc_finish(packed):
  def body(in_ref, out_ref):
    x = in_ref[...]
    sl = lax.broadcasted_iota(jnp.int32, (SBLK, SB * SBLK), 0)
    bl = lax.broadcasted_iota(jnp.int32, (SBLK, SB * SBLK), 1)
    acc = None
    for e in range(SB):
      onehot = (bl == SB * sl + e).astype(jnp.float32)
      xe = x[:, :, e * D:(e + 1) * D]
      for part in (xe,):
        z = lax.dot_general(
            part, onehot, (((0,), (0,)), ((), ())),
            preferred_element_type=jnp.float32)
        acc = z if acc is None else acc + z
    out_ref[...] = acc

  return pl.pallas_call(
      body,
      grid=(NSLAB_ALL // SBLK, HIST // HB),
      in_specs=[pl.BlockSpec((SBLK, HB, 2 * D), lambda i, j: (i, j, 0))],
      out_specs=pl.BlockSpec((HB, D, SB * SBLK), lambda i, j: (j, 0, i)),
      out_shape=jax.ShapeDtypeStruct((HIST, D, BATCH), jnp.float32),
  )(packed)


def kernel(x, w):
  packed = _sc_gather(x, w)
  out_t = _tc_finish(packed)
  return lax.stop_gradient(out_t.transpose(2, 0, 1))
